# baseline (speedup 1.0000x reference)
import functools

import jax
import jax.numpy as jnp
from jax import lax
from jax.experimental import pallas as pl
from jax.experimental.pallas import tpu as pltpu
from jax.experimental.pallas import tpu_sc as plsc

N_ROWS = 16384
N_CODES = 8192
DIM = 256
BETA_ = 1.0

BZ = 512
BW = 1024
NZ = N_ROWS // BZ
NWB = N_CODES // BW
LOSS_SCALE = (1.0 + BETA_) / (N_ROWS * DIM)


def _dist_argmin_body(z_ref, w_ref, idx_ref, loss_ref, minv, marg):
    i = pl.program_id(0)
    j = pl.program_id(1)
    z = z_ref[...]
    w = w_ref[pl.ds(j * BW, BW), :]
    zn = jnp.sum(z * z, axis=1, keepdims=True)
    wn = jnp.sum(w * w, axis=1)
    mm = lax.dot_general(z, w, dimension_numbers=(((1,), (1,)), ((), ())),
                         preferred_element_type=jnp.float32)
    dd = (zn + wn[None, :]) - 2.0 * mm
    bmin = jnp.min(dd, axis=1)
    ids = lax.broadcasted_iota(jnp.int32, dd.shape, 1)
    barg = jnp.min(jnp.where(dd == bmin[:, None], ids, jnp.int32(2**31 - 1)),
                   axis=1) + j * BW

    @pl.when(j == 0)
    def _():
        minv[...] = bmin
        marg[...] = barg

    @pl.when(j > 0)
    def _():
        cur = minv[...]
        upd = bmin < cur
        minv[...] = jnp.where(upd, bmin, cur)
        marg[...] = jnp.where(upd, barg, marg[...])

    @pl.when(j == NWB - 1)
    def _():
        idx_ref[0, 0, :] = marg[...]
        part = jnp.sum(minv[...]).reshape(1, 1)
        prev = jnp.where(i == 0, jnp.zeros((1, 1), jnp.float32), loss_ref[...])
        tot = prev + part
        loss_ref[...] = jnp.where(i == NZ - 1, tot * LOSS_SCALE, tot)


def _dist_argmin(z, W):
    return pl.pallas_call(
        _dist_argmin_body,
        grid=(NZ, NWB),
        in_specs=[
            pl.BlockSpec((BZ, DIM), lambda i, j: (i, 0)),
            pl.BlockSpec((N_CODES, DIM), lambda i, j: (0, 0)),
        ],
        out_specs=[
            pl.BlockSpec((1, 1, BZ), lambda i, j: (i, 0, 0)),
            pl.BlockSpec((1, 1), lambda i, j: (0, 0)),
        ],
        out_shape=[
            jax.ShapeDtypeStruct((NZ, 1, BZ), jnp.int32),
            jax.ShapeDtypeStruct((1, 1), jnp.float32),
        ],
        scratch_shapes=[
            pltpu.VMEM((BZ,), jnp.float32),
            pltpu.VMEM((BZ,), jnp.int32),
        ],
        compiler_params=pltpu.CompilerParams(
            dimension_semantics=("arbitrary", "arbitrary")),
    )(z, W)


_NC = 2
_NS = 16
_NWK = _NC * _NS
_BPW = N_ROWS // _NWK
_CH = 128
_NCH = _BPW // _CH


def _sc_gather(W, idx):
    mesh = plsc.VectorSubcoreMesh(core_axis_name="c", subcore_axis_name="s")

    @functools.partial(
        pl.kernel, mesh=mesh,
        out_type=jax.ShapeDtypeStruct((N_ROWS, DIM), jnp.float32),
        scratch_types=[
            pltpu.VMEM((_CH,), jnp.int32),
            pltpu.VMEM((_CH, DIM), jnp.float32),
            pltpu.SemaphoreType.DMA,
        ],
    )
    def k(table_hbm, idx_hbm, out_hbm, idx_v, rows_v, sem):
        wid = lax.axis_index("s") * _NC + lax.axis_index("c")
        base = wid * _BPW
        for c in range(_NCH):
            off = base + c * _CH
            pltpu.sync_copy(idx_hbm.at[pl.ds(off, _CH)], idx_v)
            pltpu.async_copy(table_hbm.at[idx_v], rows_v, sem).wait()
            pltpu.sync_copy(rows_v, out_hbm.at[pl.ds(off, _CH)])

    return k(W, idx)


def kernel(z, W):
    idx3, loss2 = _dist_argmin(z, W)
    idx = idx3.reshape(N_ROWS)
    z_q = _sc_gather(W, idx)
    loss = loss2[0, 0]
    return (loss, z_q, idx)


# speedup vs baseline: 1.3536x; 1.3536x over previous
import functools

import jax
import jax.numpy as jnp
from jax import lax
from jax.experimental import pallas as pl
from jax.experimental.pallas import tpu as pltpu
from jax.experimental.pallas import tpu_sc as plsc

N_ROWS = 16384
N_CODES = 8192
DIM = 256
BETA_ = 1.0

BZ = 512
BW = 1024
NZ = N_ROWS // BZ
NWB = N_CODES // BW
LOSS_SCALE = (1.0 + BETA_) / (N_ROWS * DIM)


def _dist_argmin_body(z_ref, w_ref, idx_ref, loss_ref, runmin, runj):
    i = pl.program_id(0)
    j = pl.program_id(1)
    z = z_ref[...]
    w = w_ref[pl.ds(j * BW, BW), :]
    zn = jnp.sum(z * z, axis=1, keepdims=True)
    wn = jnp.sum(w * w, axis=1)
    mm = lax.dot_general(z, w, dimension_numbers=(((1,), (1,)), ((), ())),
                         preferred_element_type=jnp.float32)
    dd = (zn + wn[None, :]) - 2.0 * mm

    @pl.when(j == 0)
    def _():
        runmin[...] = dd
        runj[...] = jnp.zeros_like(runj)

    @pl.when(j > 0)
    def _():
        cur = runmin[...]
        upd = dd < cur
        runmin[...] = jnp.where(upd, dd, cur)
        runj[...] = jnp.where(upd, j, runj[...])

    @pl.when(j == NWB - 1)
    def _():
        rm = runmin[...]
        gmin = jnp.min(rm, axis=1)
        codes = runj[...] * BW + lax.broadcasted_iota(jnp.int32, rm.shape, 1)
        idx_ref[0, 0, :] = jnp.min(
            jnp.where(rm == gmin[:, None], codes, jnp.int32(2**31 - 1)), axis=1)
        part = jnp.sum(gmin).reshape(1, 1)
        prev = jnp.where(i == 0, jnp.zeros((1, 1), jnp.float32), loss_ref[...])
        tot = prev + part
        loss_ref[...] = jnp.where(i == NZ - 1, tot * LOSS_SCALE, tot)


def _dist_argmin(z, W):
    return pl.pallas_call(
        _dist_argmin_body,
        grid=(NZ, NWB),
        in_specs=[
            pl.BlockSpec((BZ, DIM), lambda i, j: (i, 0)),
            pl.BlockSpec((N_CODES, DIM), lambda i, j: (0, 0)),
        ],
        out_specs=[
            pl.BlockSpec((1, 1, BZ), lambda i, j: (i, 0, 0)),
            pl.BlockSpec((1, 1), lambda i, j: (0, 0)),
        ],
        out_shape=[
            jax.ShapeDtypeStruct((NZ, 1, BZ), jnp.int32),
            jax.ShapeDtypeStruct((1, 1), jnp.float32),
        ],
        scratch_shapes=[
            pltpu.VMEM((BZ, BW), jnp.float32),
            pltpu.VMEM((BZ, BW), jnp.int32),
        ],
        compiler_params=pltpu.CompilerParams(
            dimension_semantics=("arbitrary", "arbitrary")),
    )(z, W)


_NC = 2
_NS = 16
_NWK = _NC * _NS
_BPW = N_ROWS // _NWK
_CH = 128
_NCH = _BPW // _CH


def _sc_gather(W, idx):
    mesh = plsc.VectorSubcoreMesh(core_axis_name="c", subcore_axis_name="s")

    @functools.partial(
        pl.kernel, mesh=mesh,
        out_type=jax.ShapeDtypeStruct((N_ROWS, DIM), jnp.float32),
        scratch_types=[
            pltpu.VMEM((_CH,), jnp.int32),
            pltpu.VMEM((_CH, DIM), jnp.float32),
            pltpu.SemaphoreType.DMA,
        ],
    )
    def k(table_hbm, idx_hbm, out_hbm, idx_v, rows_v, sem):
        wid = lax.axis_index("s") * _NC + lax.axis_index("c")
        base = wid * _BPW
        for c in range(_NCH):
            off = base + c * _CH
            pltpu.sync_copy(idx_hbm.at[pl.ds(off, _CH)], idx_v)
            pltpu.async_copy(table_hbm.at[idx_v], rows_v, sem).wait()
            pltpu.sync_copy(rows_v, out_hbm.at[pl.ds(off, _CH)])

    return k(W, idx)


def kernel(z, W):
    idx3, loss2 = _dist_argmin(z, W)
    idx = idx3.reshape(N_ROWS)
    z_q = _sc_gather(W, idx)
    loss = loss2[0, 0]
    return (loss, z_q, idx)
